# baseline (speedup 1.0000x reference)
import jax
import jax.numpy as jnp
from jax.experimental import pallas as pl
from jax.experimental.pallas import tpu as pltpu

B, N, D, G, K = 4, 1024, 128, 4, 3


def _ecopy(e_hbm_ref, eraw_ref, sem_ref, b, k):
    return pltpu.make_async_copy(
        e_hbm_ref.at[b, k],
        eraw_ref.at[b % 2, k - 1],
        sem_ref.at[b % 2, k - 1],
    )


def _start(e_hbm_ref, eraw_ref, sem_ref, b):
    _ecopy(e_hbm_ref, eraw_ref, sem_ref, b, 1).start()
    _ecopy(e_hbm_ref, eraw_ref, sem_ref, b, 2).start()


def _body(e_hbm_ref, x_ref, w1_ref, b1_ref, w2_ref, b2_ref, o_ref,
          eraw_ref, sem_ref):
    b = pl.program_id(0)

    @pl.when(b == 0)
    def _first():
        _start(e_hbm_ref, eraw_ref, sem_ref, 0)

    @pl.when(b < B - 1)
    def _prefetch_next():
        _start(e_hbm_ref, eraw_ref, sem_ref, b + 1)

    _ecopy(e_hbm_ref, eraw_ref, sem_ref, b, 1).wait()
    _ecopy(e_hbm_ref, eraw_ref, sem_ref, b, 2).wait()

    slot = b % 2
    a = (eraw_ref[slot, 0] + eraw_ref[slot, 1]).astype(jnp.bfloat16)
    agg = jnp.dot(a, x_ref[0], preferred_element_type=jnp.float32)
    h = jnp.dot(agg, w1_ref[0], preferred_element_type=jnp.float32)
    h = jnp.maximum(h + b1_ref[0], 0.0)
    agg = jnp.dot(a, h.astype(jnp.bfloat16), preferred_element_type=jnp.float32)
    h = jnp.dot(agg, w2_ref[0], preferred_element_type=jnp.float32)
    o_ref[0] = jnp.maximum(h + b2_ref[0], 0.0)


def kernel(t, x, E, W1, b1, W2, b2, interpret=False):
    et = jnp.transpose(E, (0, 3, 1, 2))
    xb = x.astype(jnp.bfloat16)
    b1r = b1.reshape(G, 1, D)
    b2r = b2.reshape(G, 1, D)
    return pl.pallas_call(
        _body,
        grid=(B,),
        in_specs=[
            pl.BlockSpec(memory_space=pltpu.MemorySpace.HBM),
            pl.BlockSpec((1, N, D), lambda b: (b, 0, 0)),
            pl.BlockSpec((1, D, D), lambda b: (b % G, 0, 0)),
            pl.BlockSpec((1, 1, D), lambda b: (b % G, 0, 0)),
            pl.BlockSpec((1, D, D), lambda b: (b % G, 0, 0)),
            pl.BlockSpec((1, 1, D), lambda b: (b % G, 0, 0)),
        ],
        out_specs=pl.BlockSpec((1, N, D), lambda b: (b, 0, 0)),
        out_shape=jax.ShapeDtypeStruct((B, N, D), jnp.float32),
        scratch_shapes=[
            pltpu.VMEM((2, 2, N, N), jnp.int32),
            pltpu.SemaphoreType.DMA((2, 2)),
        ],
        compiler_params=pltpu.CompilerParams(
            dimension_semantics=("arbitrary",),
        ),
        interpret=interpret,
    )(et, xb, W1, b1r, W2, b2r)


# speedup vs baseline: 1.0007x; 1.0007x over previous
import jax
import jax.numpy as jnp
from jax.experimental import pallas as pl
from jax.experimental.pallas import tpu as pltpu

B, N, D, G, K = 4, 1024, 128, 4, 3


def _ecopy(e_hbm_ref, eraw_ref, sem_ref, b):
    return pltpu.make_async_copy(
        e_hbm_ref.at[b, pl.ds(N, 2 * N), :],
        eraw_ref.at[b % 2],
        sem_ref.at[b % 2],
    )


def _body(e_hbm_ref, x_ref, w1_ref, b1_ref, w2_ref, b2_ref, o_ref,
          eraw_ref, sem_ref):
    b = pl.program_id(0)

    @pl.when(b == 0)
    def _first():
        _ecopy(e_hbm_ref, eraw_ref, sem_ref, 0).start()

    @pl.when(b < B - 1)
    def _prefetch_next():
        _ecopy(e_hbm_ref, eraw_ref, sem_ref, b + 1).start()

    _ecopy(e_hbm_ref, eraw_ref, sem_ref, b).wait()

    slot = b % 2
    a = (eraw_ref[slot, :N, :] + eraw_ref[slot, N:, :]).astype(jnp.bfloat16)
    agg = jnp.dot(a, x_ref[0], preferred_element_type=jnp.float32)
    h = jnp.dot(agg, w1_ref[0], preferred_element_type=jnp.float32)
    h = jnp.maximum(h + b1_ref[0], 0.0)
    agg = jnp.dot(a, h.astype(jnp.bfloat16), preferred_element_type=jnp.float32)
    h = jnp.dot(agg, w2_ref[0], preferred_element_type=jnp.float32)
    o_ref[0] = jnp.maximum(h + b2_ref[0], 0.0)


def kernel(t, x, E, W1, b1, W2, b2, interpret=False):
    et = jnp.transpose(E, (0, 3, 1, 2)).reshape(B, K * N, N)
    xb = x.astype(jnp.bfloat16)
    b1r = b1.reshape(G, 1, D)
    b2r = b2.reshape(G, 1, D)
    return pl.pallas_call(
        _body,
        grid=(B,),
        in_specs=[
            pl.BlockSpec(memory_space=pltpu.MemorySpace.HBM),
            pl.BlockSpec((1, N, D), lambda b: (b, 0, 0)),
            pl.BlockSpec((1, D, D), lambda b: (b % G, 0, 0)),
            pl.BlockSpec((1, 1, D), lambda b: (b % G, 0, 0)),
            pl.BlockSpec((1, D, D), lambda b: (b % G, 0, 0)),
            pl.BlockSpec((1, 1, D), lambda b: (b % G, 0, 0)),
        ],
        out_specs=pl.BlockSpec((1, N, D), lambda b: (b, 0, 0)),
        out_shape=jax.ShapeDtypeStruct((B, N, D), jnp.float32),
        scratch_shapes=[
            pltpu.VMEM((2, 2 * N, N), jnp.int32),
            pltpu.SemaphoreType.DMA((2,)),
        ],
        compiler_params=pltpu.CompilerParams(
            dimension_semantics=("arbitrary",),
        ),
        interpret=interpret,
    )(et, xb, W1, b1r, W2, b2r)


# speedup vs baseline: 1.1419x; 1.1411x over previous
import jax
import jax.numpy as jnp
from jax.experimental import pallas as pl
from jax.experimental.pallas import tpu as pltpu

B, N, D, G, K = 4, 1024, 128, 4, 3


def _ecopy(e_hbm_ref, eraw_ref, sem_ref, b):
    return pltpu.make_async_copy(
        e_hbm_ref.at[b, pl.ds(N, 2 * N), :],
        eraw_ref.at[b % 2],
        sem_ref.at[b % 2],
    )


def _body(e_hbm_ref, x_ref, w1_ref, b1_ref, w2_ref, b2_ref, o_ref,
          eraw_ref, sem_ref):
    b = pl.program_id(0)

    @pl.when(b == 0)
    def _first():
        _ecopy(e_hbm_ref, eraw_ref, sem_ref, 0).start()

    @pl.when(b < B - 1)
    def _prefetch_next():
        _ecopy(e_hbm_ref, eraw_ref, sem_ref, b + 1).start()

    _ecopy(e_hbm_ref, eraw_ref, sem_ref, b).wait()

    slot = b % 2
    a = (eraw_ref[slot, :N, :] + eraw_ref[slot, N:, :]).astype(jnp.bfloat16)
    agg = jnp.dot(a, x_ref[0].astype(jnp.bfloat16),
                  preferred_element_type=jnp.float32)
    h = jnp.dot(agg, w1_ref[0], preferred_element_type=jnp.float32)
    h = jnp.maximum(h + b1_ref[0], 0.0)
    agg = jnp.dot(a, h.astype(jnp.bfloat16), preferred_element_type=jnp.float32)
    h = jnp.dot(agg, w2_ref[0], preferred_element_type=jnp.float32)
    o_ref[0] = jnp.maximum(h + b2_ref[0], 0.0)


def kernel(t, x, E, W1, b1, W2, b2, interpret=False):
    et = jnp.transpose(E, (0, 3, 1, 2)).reshape(B, K * N, N)
    b1r = b1.reshape(G, 1, D)
    b2r = b2.reshape(G, 1, D)
    return pl.pallas_call(
        _body,
        grid=(B,),
        in_specs=[
            pl.BlockSpec(memory_space=pltpu.MemorySpace.HBM),
            pl.BlockSpec((1, N, D), lambda b: (b, 0, 0)),
            pl.BlockSpec((1, D, D), lambda b: (b % G, 0, 0)),
            pl.BlockSpec((1, 1, D), lambda b: (b % G, 0, 0)),
            pl.BlockSpec((1, D, D), lambda b: (b % G, 0, 0)),
            pl.BlockSpec((1, 1, D), lambda b: (b % G, 0, 0)),
        ],
        out_specs=pl.BlockSpec((1, N, D), lambda b: (b, 0, 0)),
        out_shape=jax.ShapeDtypeStruct((B, N, D), jnp.float32),
        scratch_shapes=[
            pltpu.VMEM((2, 2 * N, N), jnp.int32),
            pltpu.SemaphoreType.DMA((2,)),
        ],
        compiler_params=pltpu.CompilerParams(
            dimension_semantics=("arbitrary",),
        ),
        interpret=interpret,
    )(et, x, W1, b1r, W2, b2r)


# speedup vs baseline: 1.2235x; 1.0715x over previous
import jax
import jax.numpy as jnp
from jax.experimental import pallas as pl
from jax.experimental.pallas import tpu as pltpu

B, N, D, G, K = 4, 1024, 128, 4, 3


def _ecopy(e_hbm_ref, eraw_ref, sem_ref, b):
    return pltpu.make_async_copy(
        e_hbm_ref.at[b, pl.ds(N, 2 * N), :],
        eraw_ref.at[b % 3],
        sem_ref.at[b % 3],
    )


def _body(e_hbm_ref, x_ref, w1_ref, b1_ref, w2_ref, b2_ref, o_ref,
          eraw_ref, sem_ref):
    b = pl.program_id(0)

    @pl.when(b == 0)
    def _first():
        _ecopy(e_hbm_ref, eraw_ref, sem_ref, 0).start()
        _ecopy(e_hbm_ref, eraw_ref, sem_ref, 1).start()

    @pl.when(b < B - 2)
    def _prefetch_next():
        _ecopy(e_hbm_ref, eraw_ref, sem_ref, b + 2).start()

    _ecopy(e_hbm_ref, eraw_ref, sem_ref, b).wait()

    slot = b % 3
    a = (eraw_ref[slot, :N, :] + eraw_ref[slot, N:, :]).astype(jnp.bfloat16)
    agg = jnp.dot(a, x_ref[0].astype(jnp.bfloat16),
                  preferred_element_type=jnp.float32)
    h = jnp.dot(agg, w1_ref[0], preferred_element_type=jnp.float32)
    h = jnp.maximum(h + b1_ref[0], 0.0)
    agg = jnp.dot(a, h.astype(jnp.bfloat16), preferred_element_type=jnp.float32)
    h = jnp.dot(agg, w2_ref[0], preferred_element_type=jnp.float32)
    o_ref[0] = jnp.maximum(h + b2_ref[0], 0.0)


def kernel(t, x, E, W1, b1, W2, b2, interpret=False):
    et = jnp.transpose(E, (0, 3, 1, 2)).reshape(B, K * N, N)
    b1r = b1.reshape(G, 1, D)
    b2r = b2.reshape(G, 1, D)
    return pl.pallas_call(
        _body,
        grid=(B,),
        in_specs=[
            pl.BlockSpec(memory_space=pltpu.MemorySpace.HBM),
            pl.BlockSpec((1, N, D), lambda b: (b, 0, 0)),
            pl.BlockSpec((1, D, D), lambda b: (b % G, 0, 0)),
            pl.BlockSpec((1, 1, D), lambda b: (b % G, 0, 0)),
            pl.BlockSpec((1, D, D), lambda b: (b % G, 0, 0)),
            pl.BlockSpec((1, 1, D), lambda b: (b % G, 0, 0)),
        ],
        out_specs=pl.BlockSpec((1, N, D), lambda b: (b, 0, 0)),
        out_shape=jax.ShapeDtypeStruct((B, N, D), jnp.float32),
        scratch_shapes=[
            pltpu.VMEM((3, 2 * N, N), jnp.int32),
            pltpu.SemaphoreType.DMA((3,)),
        ],
        compiler_params=pltpu.CompilerParams(
            dimension_semantics=("arbitrary",),
        ),
        interpret=interpret,
    )(et, x, W1, b1r, W2, b2r)
